# baseline (speedup 1.0000x reference)
import functools
import math

import jax
import jax.numpy as jnp
from jax import lax
from jax.experimental import pallas as pl
from jax.experimental.pallas import tpu as pltpu
from jax.experimental.pallas import tpu_sc as plsc

B = 4096
L = 200
D = 64
DP = 128
V = 1000000
SCALE = math.sqrt(D)

NW = 32
ROWS = B * L
PER_W = ROWS // NW
BROWS_W = B // NW
TBLK = 32
LANES = 16

PK = 64
PCHUNKS = V // PK
PROUNDS = -(-PCHUNKS // NW)

HOFF = (0, 72, 136)
HSZ = (72, 64, 64)
HMAX = 72


def _pad_body(tab_hbm, tpad_hbm, ins, stages, sem_i, sem_o):
    cid = lax.axis_index("c")
    sid = lax.axis_index("s")
    wid = sid * 2 + cid

    def start_in(buf, g):
        pltpu.make_async_copy(
            tab_hbm.at[pl.ds(g * PK, PK)], ins[buf], sem_i.at[buf]
        ).start()

    def wait_in(buf, g):
        pltpu.make_async_copy(
            tab_hbm.at[pl.ds(g * PK, PK)], ins[buf], sem_i.at[buf]
        ).wait()

    def move(buf):
        src, dst = ins[buf], stages[buf]

        def row(r, carry):
            for j in range(D // LANES):
                sl = pl.ds(j * LANES, LANES)
                dst[r, sl] = src[r, sl]
            return carry

        lax.fori_loop(0, PK, row, 0)

    for buf in range(2):
        @pl.when(buf * NW + wid < PCHUNKS)
        def _(buf=buf):
            start_in(buf, buf * NW + wid)

    def outer(t, carry):
        for buf in range(2):
            rnd = t * 2 + buf
            g = rnd * NW + wid

            @pl.when(g < PCHUNKS)
            def _():
                wait_in(buf, g)
                @pl.when(rnd >= 2)
                def _():
                    pltpu.make_async_copy(
                        stages[buf],
                        tpad_hbm.at[pl.ds((g - 2 * NW) * PK, PK)],
                        sem_o.at[buf],
                    ).wait()

                move(buf)

                @pl.when(g + 2 * NW < PCHUNKS)
                def _():
                    start_in(buf, g + 2 * NW)

                pltpu.make_async_copy(
                    stages[buf], tpad_hbm.at[pl.ds(g * PK, PK)], sem_o.at[buf]
                ).start()
        return carry

    lax.fori_loop(0, (PROUNDS + 1) // 2, outer, 0)

    nr = jnp.where(wid < PCHUNKS - (PROUNDS - 1) * NW, PROUNDS, PROUNDS - 1)
    for buf in range(2):
        last = nr - 1 - lax.rem(nr - 1 - buf + 2, 2)
        @pl.when(last >= 0)
        def _(buf=buf, last=last):
            pltpu.make_async_copy(
                stages[buf],
                tpad_hbm.at[pl.ds((last * NW + wid) * PK, PK)],
                sem_o.at[buf],
            ).wait()


def _gather_body(tok_hbm, tpad_hbm, out_hbm, tok_v, idx_v, ins, outs,
                 sem_g, sem_s):
    cid = lax.axis_index("c")
    sid = lax.axis_index("s")
    wid = sid * 2 + cid
    bbase = wid * BROWS_W

    offs = [16 * k for k in range(12)] + [184]

    for blk in range(BROWS_W // TBLK):
        pltpu.sync_copy(tok_hbm.at[pl.ds(bbase + blk * TBLK, TBLK)], tok_v)

        def row(r, carry, blk=blk):
            for o in offs:
                idx_v[pl.ds((blk * TBLK + r) * L + o, LANES)] = (
                    tok_v[r, pl.ds(o, LANES)]
                )
            return carry

        lax.fori_loop(0, TBLK, row, 0)

    def start_gather(h, bb):
        pltpu.make_async_copy(
            tpad_hbm.at[idx_v.at[pl.ds(bb * L + HOFF[h], HSZ[h])]],
            ins[h].at[pl.ds(0, HSZ[h])],
            sem_g.at[h],
        ).start()

    def wait_gather(h, bb):
        pltpu.make_async_copy(
            tpad_hbm.at[idx_v.at[pl.ds(bb * L + HOFF[h], HSZ[h])]],
            ins[h].at[pl.ds(0, HSZ[h])],
            sem_g.at[h],
        ).wait()

    def out_dma(h, bb):
        return pltpu.make_async_copy(
            outs[h].at[pl.ds(0, HSZ[h])],
            out_hbm.at[bbase + bb, pl.ds(HOFF[h], HSZ[h])],
            sem_s.at[h],
        )

    def scale(h):
        src, dst = ins[h], outs[h]

        def rowblk(i, carry):
            r0 = i * 8
            for rr in range(8):
                for j in range(D // LANES):
                    sl = pl.ds(j * LANES, LANES)
                    dst[r0 + rr, sl] = src[r0 + rr, sl] * jnp.float32(SCALE)
            return carry

        lax.fori_loop(0, HSZ[h] // 8, rowblk, 0)

    for h in range(3):
        start_gather(h, 0)

    def outer(bb, carry):
        for h in range(3):
            wait_gather(h, bb)
            @pl.when(bb >= 1)
            def _():
                out_dma(h, bb - 1).wait()

            scale(h)

            @pl.when(bb + 1 < BROWS_W)
            def _():
                start_gather(h, bb + 1)

            out_dma(h, bb).start()
        return carry

    lax.fori_loop(0, BROWS_W, outer, 0)

    for h in range(3):
        out_dma(h, BROWS_W - 1).wait()


_pad = functools.partial(
    pl.kernel,
    mesh=plsc.VectorSubcoreMesh(core_axis_name="c", subcore_axis_name="s"),
    out_type=jax.ShapeDtypeStruct((V, DP), jnp.float32),
    scratch_types=[
        [pltpu.VMEM((PK, D), jnp.float32) for _ in range(2)],
        [pltpu.VMEM((PK, DP), jnp.float32) for _ in range(2)],
        pltpu.SemaphoreType.DMA((2,)),
        pltpu.SemaphoreType.DMA((2,)),
    ],
)(_pad_body)


_gather = functools.partial(
    pl.kernel,
    mesh=plsc.VectorSubcoreMesh(core_axis_name="c", subcore_axis_name="s"),
    out_type=jax.ShapeDtypeStruct((B, L, D), jnp.float32),
    scratch_types=[
        pltpu.VMEM((TBLK, L), jnp.int32),
        pltpu.VMEM((PER_W,), jnp.int32),
        [pltpu.VMEM((HMAX, DP), jnp.float32) for _ in range(3)],
        [pltpu.VMEM((HMAX, D), jnp.float32) for _ in range(3)],
        pltpu.SemaphoreType.DMA((3,)),
        pltpu.SemaphoreType.DMA((3,)),
    ],
)(_gather_body)


def kernel(tokens, table):
    tpad = _pad(table)
    return _gather(tokens, tpad)
